# baseline (speedup 1.0000x reference)
import jax
import jax.numpy as jnp
from jax import lax
from jax.experimental import pallas as pl
from jax.experimental.pallas import tpu as pltpu
from jax.experimental.pallas import tpu_sc as plsc

N_W = 8000
N_S = 2000
E = 160000
D = 256

NC = 2
NS = 16
NT = NC * NS
ACC_ROWS = 2048
RPT = ACC_ROWS // NT
SCB = 2048
EP = 163840
NBLK = EP // SCB
LIST = 8192
CHUNK = 64
NBUF = 4

_GDN = lax.GatherDimensionNumbers(offset_dims=(), collapsed_slice_dims=(0,),
                                  start_index_map=(0,))


def _splat(vec, lane_idx):
    return lax.gather(vec, lane_idx, _GDN, (1,),
                      mode=lax.GatherScatterMode.PROMISE_IN_BOUNDS)


def _sc_body(feat_hbm, key_hbm, zero_hbm, out_hbm,
             kblk0, kblk1, fsrc, floc, rows0, rows1, rows2, rows3, acc,
             sem_k0, sem_k1, sem_r0, sem_r1, sem_r2, sem_r3):
    cid = lax.axis_index("c")
    sid = lax.axis_index("s")
    gid = sid * NC + cid

    pltpu.sync_copy(zero_hbm, acc)
    iota16 = lax.iota(jnp.int32, 16)

    kblks = (kblk0, kblk1)
    ksems = (sem_k0, sem_k1)
    pltpu.async_copy(key_hbm.at[pl.ds(0, SCB)], kblk0, sem_k0)

    UNROLL = 4

    def scan_vec_mk(kblk):
        def scan_vec(v, cnt_splat):
            kvs = [kblk[pl.ds((v * UNROLL + u) * 16, 16)] for u in range(UNROLL)]
            ms = [lax.shift_right_logical(kv, 19) == gid for kv in kvs]
            mis = [m.astype(jnp.int32) for m in ms]
            prefs = [jnp.cumsum(mi) for mi in mis]
            pops = [plsc.all_reduce_population_count(m) for m in ms]
            for u in range(UNROLL):
                offs = cnt_splat + prefs[u] - mis[u]
                sv = jnp.bitwise_and(kvs[u], 8191)
                lv = jnp.bitwise_and(lax.shift_right_logical(kvs[u], 5),
                                     64 * 256 - 256)
                plsc.store_scatter(fsrc, [offs], sv, mask=ms[u])
                plsc.store_scatter(floc, [offs], lv, mask=ms[u])
                cnt_splat = cnt_splat + pops[u]
            return cnt_splat
        return scan_vec

    def scan_blk2(b2, cnt_splat):
        for p in range(2):
            blk = b2 * 2 + p
            pltpu.make_async_copy(key_hbm.at[pl.ds(0, SCB)],
                                  kblks[p], ksems[p]).wait()
            nxt = pl.multiple_of((blk + 1) * SCB, SCB)

            @pl.when(blk + 1 < NBLK)
            def _():
                pltpu.async_copy(key_hbm.at[pl.ds(nxt, SCB)],
                                 kblks[1 - p], ksems[1 - p])

            cnt_splat = lax.fori_loop(0, SCB // (16 * UNROLL),
                                      scan_vec_mk(kblks[p]), cnt_splat)
            cnt_splat = jnp.minimum(cnt_splat, LIST - 256)
        return cnt_splat

    cnt_splat = lax.fori_loop(0, NBLK // 2, scan_blk2,
                              jnp.zeros((16,), jnp.int32))
    cnt = jnp.max(cnt_splat)

    zvec = jnp.zeros((16,), jnp.int32)
    dvec = jnp.full((16,), RPT * D, jnp.int32)
    for k in range(9):
        tidx = cnt + k * 16 + iota16
        tm = tidx < LIST
        plsc.store_scatter(fsrc, [tidx], zvec, mask=tm)
        plsc.store_scatter(floc, [tidx], dvec, mask=tm)

    nchunks = (cnt + CHUNK - 1) // CHUNK
    rowss = (rows0, rows1, rows2, rows3)
    rsems = (sem_r0, sem_r1, sem_r2, sem_r3)
    col_const = [iota16 + g * 16 for g in range(D // 16)]
    lane_idx = [jnp.full((16, 1), lane, jnp.int32) for lane in range(16)]

    for b in range(NBUF - 1):
        @pl.when(b < nchunks)
        def _():
            pltpu.async_copy(feat_hbm.at[fsrc.at[pl.ds(b * CHUNK, CHUNK)]],
                             rowss[b], rsems[b])

    def chunk_ring_body(cr, carry):
        for p in range(NBUF):
            ch = cr * NBUF + p
            rows, sem = rowss[p], rsems[p]

            @pl.when(ch < nchunks)
            def _():
                nxt = ch + NBUF - 1
                nb = (p + NBUF - 1) % NBUF

                @pl.when(nxt < nchunks)
                def _():
                    pltpu.async_copy(
                        feat_hbm.at[fsrc.at[pl.ds(nxt * CHUNK, CHUNK)]],
                        rowss[nb], rsems[nb])

                pltpu.make_async_copy(feat_hbm.at[pl.ds(0, CHUNK)],
                                      rows, sem).wait()

                def group_body(j, carry2):
                    flocv = floc[pl.ds(ch * CHUNK + j * 16, 16)]
                    for lane in range(16):
                        l = j * 16 + lane
                        r256 = _splat(flocv, lane_idx[lane])
                        for g in range(D // 16):
                            addr = r256 + col_const[g]
                            plsc.addupdate_scatter(acc, [addr],
                                                   rows[l, pl.ds(g * 16, 16)])
                    return carry2

                lax.fori_loop(0, CHUNK // 16, group_body, None)
        return carry

    lax.fori_loop(0, (nchunks + NBUF - 1) // NBUF, chunk_ring_body, None)

    pltpu.sync_copy(acc.at[pl.ds(0, RPT * D)],
                    out_hbm.at[pl.ds(gid * (RPT * D), RPT * D)])


_SC_SCRATCH = [
    pltpu.VMEM((SCB,), jnp.int32),
    pltpu.VMEM((SCB,), jnp.int32),
    pltpu.VMEM((LIST,), jnp.int32),
    pltpu.VMEM((LIST,), jnp.int32),
    pltpu.VMEM((CHUNK, D), jnp.float32),
    pltpu.VMEM((CHUNK, D), jnp.float32),
    pltpu.VMEM((CHUNK, D), jnp.float32),
    pltpu.VMEM((CHUNK, D), jnp.float32),
    pltpu.VMEM((RPT * D + D,), jnp.float32),
    pltpu.SemaphoreType.DMA,
    pltpu.SemaphoreType.DMA,
    pltpu.SemaphoreType.DMA,
    pltpu.SemaphoreType.DMA,
    pltpu.SemaphoreType.DMA,
    pltpu.SemaphoreType.DMA,
]

_sc_segment_sum = pl.kernel(
    _sc_body,
    out_type=jax.ShapeDtypeStruct((ACC_ROWS * D,), jnp.float32),
    mesh=plsc.VectorSubcoreMesh(core_axis_name="c", subcore_axis_name="s"),
    compiler_params=pltpu.CompilerParams(needs_layout_passes=False),
    scratch_types=_SC_SCRATCH,
)


def _tc_linear_body(p_ref, w_ref, b_ref, o_ref):
    o_ref[...] = lax.dot_general(
        p_ref[...], w_ref[...], (((1,), (1,)), ((), ())),
        preferred_element_type=jnp.float32) + b_ref[...]


_tc_linear = pl.pallas_call(
    _tc_linear_body,
    out_shape=jax.ShapeDtypeStruct((ACC_ROWS, D), jnp.float32),
)


def kernel(feature, src_idx, dst_idx, W, b):
    pad = EP - E
    src_p = jnp.concatenate([src_idx, jnp.zeros((pad,), jnp.int32)])
    dst_p = jnp.concatenate([dst_idx,
                             jnp.full((pad,), ACC_ROWS - 1, jnp.int32)])
    keys = dst_p * 8192 + src_p
    zeros = jnp.zeros((RPT * D + D,), jnp.float32)
    h = _sc_segment_sum(feature, keys, zeros).reshape(ACC_ROWS, D)
    out = _tc_linear(h, W, b.reshape(1, D))
    return out[:N_S]
